# baseline (speedup 1.0000x reference)
import functools

import jax
import jax.numpy as jnp
from jax import lax
from jax.experimental import pallas as pl
from jax.experimental.pallas import tpu as pltpu
from jax.experimental.pallas import tpu_sc as plsc

ALPHA = 0.5
BETA = 0.5
EPS = 1e-06
C = 19
B = 8
HW = 512 * 512
P = B * HW
LOG2_SLAB = 18

CH_ROWS = 64
CH = CH_ROWS * 128


def _sc_partials(nw):
    pw = P // nw
    rows_pw = pw // 128
    nch = pw // CH
    w_per_b = HW // pw

    mesh = plsc.VectorSubcoreMesh(core_axis_name="c", subcore_axis_name="s")

    @functools.partial(
        pl.kernel,
        out_type=[
            jax.ShapeDtypeStruct((nw, 16), jnp.float32),
            jax.ShapeDtypeStruct((nw, 16), jnp.float32),
        ],
        mesh=mesh,
        scratch_types=[
            pltpu.VMEM((CH_ROWS, 128), jnp.int32),
            pltpu.VMEM((CH,), jnp.int32),
            pltpu.VMEM((CH,), jnp.float32),
            pltpu.VMEM((16,), jnp.float32),
            pltpu.VMEM((16,), jnp.float32),
            pltpu.SemaphoreType.DMA,
        ],
    )
    def body(yp_ref, yt_ref, outg_ref, outs_ref, lab_v, idx_v, val_v, stg_g,
             stg_s, sem):
        nc = jax.lax.axis_size("c")
        wid = lax.axis_index("s") * nc + lax.axis_index("c")
        row0 = wid * rows_pw
        b = wid // w_per_b
        pixbase = row0 * 128 + b * (C - 1) * HW

        iota = lax.iota(jnp.int32, 16)
        lane_base = [iota + (l * 16) for l in range(8)]

        zf = jnp.zeros((16,), jnp.float32)
        zi = jnp.zeros((16,), jnp.int32)
        g_acc = [zf] * 8
        s_acc = zi

        for k in range(nch):
            pltpu.sync_copy(yt_ref.at[pl.ds(row0 + k * CH_ROWS, CH_ROWS)],
                            lab_v)
            chunk_pix = pixbase + k * CH

            def idx_body(j, s_car):
                rowpix = chunk_pix + j * 128
                for l in range(8):
                    c16 = lab_v[j, pl.ds(l * 16, 16)]
                    s_car = s_car + c16
                    idx16 = (c16 << LOG2_SLAB) + (lane_base[l] + rowpix)
                    idx_v[pl.ds(j * 128 + l * 16, 16)] = idx16
                return s_car

            s_acc = lax.fori_loop(0, CH_ROWS, idx_body, s_acc)

            pltpu.async_copy(yp_ref.at[idx_v], val_v, sem).wait()

            def sum_body(j, cars):
                return tuple(
                    cars[l] + val_v[pl.ds(j * 128 + l * 16, 16)]
                    for l in range(8))

            g_acc = list(lax.fori_loop(0, CH_ROWS, sum_body, tuple(g_acc)))

        g_tot = ((g_acc[0] + g_acc[1]) + (g_acc[2] + g_acc[3])) + (
            (g_acc[4] + g_acc[5]) + (g_acc[6] + g_acc[7]))
        stg_g[...] = g_tot
        stg_s[...] = s_acc.astype(jnp.float32)
        pltpu.sync_copy(stg_g, outg_ref.at[wid])
        pltpu.sync_copy(stg_s, outs_ref.at[wid])

    return body


def _combine_kernel(g_ref, s_ref, o_ref):
    g = jnp.sum(g_ref[...])
    s = jnp.sum(s_ref[...])
    denom = 10.0 * s + BETA * (float(P) - g) + EPS
    o_ref[0, 0] = 1.0 - s / denom


def kernel(y_pred, y_true):
    info = plsc.get_sparse_core_info()
    nw = info.num_cores * info.num_subcores

    yp_flat = y_pred.reshape(-1)
    yt2 = y_true.reshape(P // 128, 128)

    gpart, spart = _sc_partials(nw)(yp_flat, yt2)

    out = pl.pallas_call(
        _combine_kernel,
        out_shape=jax.ShapeDtypeStruct((1, 1), jnp.float32),
        out_specs=pl.BlockSpec(memory_space=pltpu.SMEM),
    )(gpart, spart)
    return out.reshape(())


# speedup vs baseline: 2.0181x; 2.0181x over previous
import functools

import jax
import jax.numpy as jnp
from jax import lax
from jax.experimental import pallas as pl
from jax.experimental.pallas import tpu as pltpu
from jax.experimental.pallas import tpu_sc as plsc

ALPHA = 0.5
BETA = 0.5
EPS = 1e-06
C = 19
B = 8
HW = 512 * 512
P = B * HW

WIN_ROWS = 8
WIN = WIN_ROWS * 512
NWIN = HW // WIN
NBUF = 4


def _sc_partials(nw):
    ncg = nw // B
    nslab = -(-C // ncg)

    mesh = plsc.VectorSubcoreMesh(core_axis_name="c", subcore_axis_name="s")

    @functools.partial(
        pl.kernel,
        out_type=[
            jax.ShapeDtypeStruct((nw, 16), jnp.float32),
            jax.ShapeDtypeStruct((nw, 16), jnp.float32),
        ],
        mesh=mesh,
        compiler_params=pltpu.CompilerParams(needs_layout_passes=False),
        scratch_types=(
            [pltpu.VMEM((WIN_ROWS, 512), jnp.int32) for _ in range(NBUF)]
            + [pltpu.VMEM((nslab * WIN_ROWS, 512), jnp.float32)
               for _ in range(NBUF)]
            + [pltpu.VMEM((16,), jnp.float32), pltpu.VMEM((16,), jnp.float32)]
            + [pltpu.SemaphoreType.DMA for _ in range(NBUF)]
        ),
    )
    def body(yp_ref, yt_ref, outg_ref, outs_ref, *refs):
        labs = refs[:NBUF]
        preds = refs[NBUF:2 * NBUF]
        stg_g, stg_s = refs[2 * NBUF], refs[2 * NBUF + 1]
        sems = refs[2 * NBUF + 2:]

        nc = jax.lax.axis_size("c")
        wid = lax.axis_index("s") * nc + lax.axis_index("c")
        b = wid // ncg
        cg = wid % ncg
        c_lo = cg * nslab
        lab_row0 = b * 512

        iota = lax.iota(jnp.int32, 16)

        def start(w, p):
            rb = w * WIN_ROWS
            pltpu.async_copy(
                yt_ref.at[pl.ds(lab_row0 + rb, WIN_ROWS)], labs[p], sems[p])
            for j in range(nslab):
                c_src = jnp.minimum(c_lo + j, C - 1)
                row = (b * C + c_src) * 512 + rb
                pltpu.async_copy(
                    yp_ref.at[pl.ds(row, WIN_ROWS)],
                    preds[p].at[pl.ds(j * WIN_ROWS, WIN_ROWS)], sems[p])

        def wait(p):
            pltpu.make_async_copy(
                yp_ref.at[pl.ds(0, nslab * WIN_ROWS)], preds[p],
                sems[p]).wait()
            pltpu.make_async_copy(
                yt_ref.at[pl.ds(0, WIN_ROWS)], labs[p], sems[p]).wait()

        def compute(p, car):
            lab_ref, pred_ref = labs[p], preds[p]

            def row_body(r, c5):
                s_acc = c5[0]
                g = list(c5[1:])
                for l in range(32):
                    lab16 = lab_ref[r, pl.ds(l * 16, 16)]
                    jrel = lab16 - c_lo
                    mask = (jrel >= 0) & (jrel < nslab)
                    jsel = jnp.where(mask, jrel, 0)
                    row16 = (jsel << 3) + r
                    col16 = iota + (l * 16)
                    v16 = plsc.load_gather(pred_ref, [row16, col16])
                    g[l % 4] = g[l % 4] + jnp.where(mask, v16, 0.0)
                    s_acc = s_acc + lab16
                return (s_acc, g[0], g[1], g[2], g[3])

            return lax.fori_loop(0, WIN_ROWS, row_body, car)

        for p in range(NBUF):
            start(p, p)

        zf = jnp.zeros((16,), jnp.float32)
        car = (jnp.zeros((16,), jnp.int32), zf, zf, zf, zf)

        def outer(w4, car):
            for p in range(NBUF):
                w = w4 * NBUF + p
                wait(p)
                car = compute(p, car)

                @pl.when(w + NBUF < NWIN)
                def _():
                    start(w + NBUF, p)
            return car

        car = lax.fori_loop(0, NWIN // NBUF, outer, car)

        s_acc, g0, g1, g2, g3 = car
        g_tot = (g0 + g1) + (g2 + g3)
        cg_v = jnp.full((16,), cg, jnp.int32)
        s_fin = jnp.where(cg_v == 0, s_acc, 0).astype(jnp.float32)
        stg_g[...] = g_tot
        stg_s[...] = s_fin
        pltpu.sync_copy(stg_g, outg_ref.at[wid])
        pltpu.sync_copy(stg_s, outs_ref.at[wid])

    return body


def _combine_kernel(g_ref, s_ref, o_ref):
    g = jnp.sum(g_ref[...])
    s = jnp.sum(s_ref[...])
    denom = 10.0 * s + BETA * (float(P) - g) + EPS
    o_ref[0, 0] = 1.0 - s / denom


def kernel(y_pred, y_true):
    info = plsc.get_sparse_core_info()
    nw = info.num_cores * info.num_subcores

    yp2 = y_pred.reshape(B * C * 512, 512)
    yt2 = y_true.reshape(B * 512, 512)

    gpart, spart = _sc_partials(nw)(yp2, yt2)

    out = pl.pallas_call(
        _combine_kernel,
        out_shape=jax.ShapeDtypeStruct((1, 1), jnp.float32),
        out_specs=pl.BlockSpec(memory_space=pltpu.SMEM),
    )(gpart, spart)
    return out.reshape(())


# speedup vs baseline: 2.3653x; 1.1721x over previous
import functools

import jax
import jax.numpy as jnp
from jax import lax
from jax.experimental import pallas as pl
from jax.experimental.pallas import tpu as pltpu
from jax.experimental.pallas import tpu_sc as plsc

ALPHA = 0.5
BETA = 0.5
EPS = 1e-06
C = 19
B = 8
HW = 512 * 512
P = B * HW

WIN_ROWS = 8
WIN = WIN_ROWS * 512
NWIN = HW // WIN
NBUF = 4


def _sc_partials(nw):
    ncg = nw // B
    nslab = -(-C // ncg)

    mesh = plsc.VectorSubcoreMesh(core_axis_name="c", subcore_axis_name="s")

    @functools.partial(
        pl.kernel,
        out_type=[
            jax.ShapeDtypeStruct((nw, 16), jnp.float32),
            jax.ShapeDtypeStruct((nw, 16), jnp.float32),
        ],
        mesh=mesh,
        compiler_params=pltpu.CompilerParams(needs_layout_passes=False),
        scratch_types=(
            [pltpu.VMEM((WIN_ROWS, 512), jnp.int32) for _ in range(NBUF)]
            + [pltpu.VMEM(((nslab + 1) * WIN_ROWS, 512), jnp.float32)
               for _ in range(NBUF)]
            + [pltpu.VMEM((16,), jnp.float32), pltpu.VMEM((16,), jnp.float32)]
            + [pltpu.SemaphoreType.DMA for _ in range(NBUF)]
        ),
    )
    def body(yp_ref, yt_ref, outg_ref, outs_ref, *refs):
        labs = refs[:NBUF]
        preds = refs[NBUF:2 * NBUF]
        stg_g, stg_s = refs[2 * NBUF], refs[2 * NBUF + 1]
        sems = refs[2 * NBUF + 2:]

        nc = jax.lax.axis_size("c")
        wid = lax.axis_index("s") * nc + lax.axis_index("c")
        b = wid // ncg
        cg = wid % ncg
        c_lo = cg * nslab
        lab_row0 = b * 512

        iota = lax.iota(jnp.int32, 16)

        def start(w, p):
            rb = w * WIN_ROWS
            pltpu.async_copy(
                yt_ref.at[pl.ds(lab_row0 + rb, WIN_ROWS)], labs[p], sems[p])
            for j in range(nslab):
                c_src = jnp.minimum(c_lo + j, C - 1)
                row = (b * C + c_src) * 512 + rb
                pltpu.async_copy(
                    yp_ref.at[pl.ds(row, WIN_ROWS)],
                    preds[p].at[pl.ds(j * WIN_ROWS, WIN_ROWS)], sems[p])

        def wait(p):
            pltpu.make_async_copy(
                yp_ref.at[pl.ds(0, nslab * WIN_ROWS)],
                preds[p].at[pl.ds(0, nslab * WIN_ROWS)], sems[p]).wait()
            pltpu.make_async_copy(
                yt_ref.at[pl.ds(0, WIN_ROWS)], labs[p], sems[p]).wait()

        def compute(p, car):
            lab_ref, pred_ref = labs[p], preds[p]

            def row_body(r, c5):
                s_acc = c5[0]
                g = list(c5[1:])
                for l in range(32):
                    lab16 = lab_ref[r, pl.ds(l * 16, 16)]
                    jrel = plsc.bitcast(lab16 - c_lo, jnp.uint32)
                    jsel = plsc.bitcast(
                        jnp.minimum(jrel, jnp.uint32(nslab)), jnp.int32)
                    row16 = (jsel << 3) + r
                    col16 = iota + (l * 16)
                    v16 = plsc.load_gather(pred_ref, [row16, col16])
                    g[l % 4] = g[l % 4] + v16
                    s_acc = s_acc + lab16
                return (s_acc, g[0], g[1], g[2], g[3])

            return lax.fori_loop(0, WIN_ROWS, row_body, car)

        def zero_junk(p):
            zf = jnp.zeros((16,), jnp.float32)

            def zrow(r, _):
                for l in range(32):
                    preds[p][nslab * WIN_ROWS + r, pl.ds(l * 16, 16)] = zf
                return 0

            lax.fori_loop(0, WIN_ROWS, zrow, 0)

        for p in range(NBUF):
            zero_junk(p)
            start(p, p)

        zf = jnp.zeros((16,), jnp.float32)
        car = (jnp.zeros((16,), jnp.int32), zf, zf, zf, zf)

        def outer(w4, car):
            for p in range(NBUF):
                w = w4 * NBUF + p
                wait(p)
                car = compute(p, car)

                @pl.when(w + NBUF < NWIN)
                def _():
                    start(w + NBUF, p)
            return car

        car = lax.fori_loop(0, NWIN // NBUF, outer, car)

        s_acc, g0, g1, g2, g3 = car
        g_tot = (g0 + g1) + (g2 + g3)
        cg_v = jnp.full((16,), cg, jnp.int32)
        s_fin = jnp.where(cg_v == 0, s_acc, 0).astype(jnp.float32)
        stg_g[...] = g_tot
        stg_s[...] = s_fin
        pltpu.sync_copy(stg_g, outg_ref.at[wid])
        pltpu.sync_copy(stg_s, outs_ref.at[wid])

    return body


def _combine_kernel(g_ref, s_ref, o_ref):
    g = jnp.sum(g_ref[...])
    s = jnp.sum(s_ref[...])
    denom = 10.0 * s + BETA * (float(P) - g) + EPS
    o_ref[0, 0] = 1.0 - s / denom


def kernel(y_pred, y_true):
    info = plsc.get_sparse_core_info()
    nw = info.num_cores * info.num_subcores

    yp2 = y_pred.reshape(B * C * 512, 512)
    yt2 = y_true.reshape(B * 512, 512)

    gpart, spart = _sc_partials(nw)(yp2, yt2)

    out = pl.pallas_call(
        _combine_kernel,
        out_shape=jax.ShapeDtypeStruct((1, 1), jnp.float32),
        out_specs=pl.BlockSpec(memory_space=pltpu.SMEM),
    )(gpart, spart)
    return out.reshape(())
